# baseline (speedup 1.0000x reference)
import functools

import jax
import jax.numpy as jnp
from jax.experimental import pallas as pl

E = 8
D = 1024
H = 512
BT = 256


def _moe_body(x_ref, wr_ref, w1c_ref, w2_ref, gamma_ref, beta_ref, o_ref):
    x = x_ref[...]
    logits = jax.lax.dot_general(
        x, wr_ref[...], (((1,), (1,)), ((), ())),
        preferred_element_type=jnp.float32)
    iota_e = jax.lax.broadcasted_iota(jnp.int32, (BT, E), 1)
    l0 = jnp.max(logits, axis=1, keepdims=True)
    e0 = jnp.min(jnp.where(logits == l0, iota_e, E), axis=1, keepdims=True)
    masked = jnp.where(iota_e == e0, -jnp.inf, logits)
    l1 = jnp.max(masked, axis=1, keepdims=True)
    e1 = jnp.min(jnp.where(masked == l1, iota_e, E), axis=1, keepdims=True)
    c0 = jax.nn.sigmoid(l0 - l1)
    c1 = 1.0 - c0
    coef = jnp.where(iota_e == e0, c0, 0.0) + jnp.where(iota_e == e1, c1, 0.0)

    h = jax.lax.dot_general(
        x.astype(jnp.bfloat16), w1c_ref[...], (((1,), (1,)), ((), ())),
        preferred_element_type=jnp.float32)
    h = h * jax.nn.sigmoid(h)
    sel = (jax.lax.broadcasted_iota(jnp.int32, (E, E * H), 1) // H ==
           jax.lax.broadcasted_iota(jnp.int32, (E, E * H), 0)
           ).astype(jnp.float32)
    q = jax.lax.dot_general(
        coef, sel, (((1,), (0,)), ((), ())),
        preferred_element_type=jnp.float32)
    hq = (h * q).astype(jnp.bfloat16)
    y = jax.lax.dot_general(
        hq, w2_ref[...], (((1,), (0,)), ((), ())),
        preferred_element_type=jnp.float32)

    mean = jnp.mean(y, axis=-1, keepdims=True)
    var = jnp.mean((y - mean) ** 2, axis=-1, keepdims=True)
    normed = (y - mean) * jax.lax.rsqrt(var + 1e-5)
    o_ref[...] = normed * gamma_ref[...][None, :] + beta_ref[...][None, :]


@functools.partial(jax.jit, static_argnames=())
def _moe(x_flat, Wr, w1c, w2, gamma, beta):
    T = x_flat.shape[0]
    grid = (T // BT,)
    return pl.pallas_call(
        _moe_body,
        grid=grid,
        in_specs=[
            pl.BlockSpec((BT, D), lambda i: (i, 0)),
            pl.BlockSpec((E, D), lambda i: (0, 0)),
            pl.BlockSpec((E * H, D), lambda i: (0, 0)),
            pl.BlockSpec((E * H, D), lambda i: (0, 0)),
            pl.BlockSpec((D,), lambda i: (0,)),
            pl.BlockSpec((D,), lambda i: (0,)),
        ],
        out_specs=pl.BlockSpec((BT, D), lambda i: (i, 0)),
        out_shape=jax.ShapeDtypeStruct((T, D), jnp.float32),
    )(x_flat, Wr, w1c, w2, gamma, beta)


def kernel(x, Wr, w1, w2, gamma, beta):
    B, S, Dm = x.shape
    x_flat = x.reshape(-1, Dm)
    w1c = w1.reshape(E * H, Dm).astype(jnp.bfloat16)
    w2t = jnp.transpose(w2, (0, 2, 1)).reshape(E * H, Dm).astype(jnp.bfloat16)
    out = _moe(x_flat, Wr, w1c, w2t, gamma, beta)
    return (out.reshape(B, S, Dm), jnp.asarray(0.0, dtype=jnp.float32))
